# Initial kernel scaffold; baseline (speedup 1.0000x reference)
import jax
import jax.numpy as jnp
from jax.experimental import pallas as pl


def kernel(x, edge_index, W1, b1, W2, b2):
    raise NotImplementedError("write your pallas kernel here")


# speedup vs baseline: 16.7243x; 16.7243x over previous
import functools

import jax
import jax.numpy as jnp
from jax import lax
from jax.experimental import pallas as pl
from jax.experimental.pallas import tpu as pltpu
from jax.experimental.pallas import tpu_sc as plsc

CHUNK = 128
BM = 1024


def _make_sc_scatter(n_pad, nchunk, feat, with_gather, nc, ns):
    nw = nc * ns
    rows16 = n_pad // ns
    n_init = rows16 // CHUNK
    mesh = plsc.VectorSubcoreMesh(core_axis_name="c", subcore_axis_name="s")
    fpl = feat // 16

    def body(*refs):
        if with_gather:
            (table_hbm, src_hbm, dst_hbm, out_hbm,
             src_v, dst_v, rows_v, bounce_v, acc_sh, sem) = refs
        else:
            (dst_hbm, out_hbm,
             dst_v, rows_v, bounce_v, acc_sh, sem) = refs
        cid = lax.axis_index("c")
        sid = lax.axis_index("s")
        wid = sid * nc + cid

        zero = jnp.zeros((16,), jnp.float32)

        def zfill(i, carry):
            for jj in range(fpl):
                rows_v[i, pl.ds(jj * 16, 16)] = zero
            return carry

        lax.fori_loop(0, CHUNK, zfill, 0)
        for c in range(n_init):
            pltpu.sync_copy(rows_v,
                            acc_sh.at[pl.ds(sid * rows16 + c * CHUNK, CHUNK)])
        plsc.subcore_barrier()

        pltpu.sync_copy(dst_hbm.at[wid], dst_v)
        if with_gather:
            pltpu.sync_copy(src_hbm.at[wid], src_v)
        else:
            one = jnp.ones((16,), jnp.float32)

            def ofill(i, carry):
                for jj in range(fpl):
                    rows_v[i, pl.ds(jj * 16, 16)] = one
                return carry

            lax.fori_loop(0, CHUNK, ofill, 0)

        def edge_chunk(j, carry):
            if with_gather:
                pltpu.async_copy(table_hbm.at[src_v.at[j]], rows_v, sem).wait()
            pltpu.sync_copy(rows_v, acc_sh.at[dst_v.at[j]], add=True)
            return carry

        lax.fori_loop(0, nchunk, edge_chunk, 0)
        plsc.subcore_barrier()

        base = sid * rows16
        pltpu.sync_copy(acc_sh.at[pl.ds(base, rows16)], bounce_v)
        pltpu.sync_copy(bounce_v, out_hbm.at[cid, pl.ds(base, rows16)])

    scratch = [
        pltpu.VMEM((nchunk, CHUNK), jnp.int32),
        pltpu.VMEM((CHUNK, feat), jnp.float32),
        pltpu.VMEM((rows16, feat), jnp.float32),
        pltpu.VMEM_SHARED((n_pad, feat), jnp.float32),
        pltpu.SemaphoreType.DMA,
    ]
    if with_gather:
        scratch.insert(0, pltpu.VMEM((nchunk, CHUNK), jnp.int32))

    return pl.kernel(
        body,
        mesh=mesh,
        out_type=jax.ShapeDtypeStruct((nc, n_pad, feat), jnp.float32),
        scratch_types=scratch,
        compiler_params=pltpu.CompilerParams(use_tc_tiling_on_sc=False),
    )


def _tc_layer1(n_pad, k_pad):
    def body(deg_ref, x_ref, w1_ref, g1_ref):
        deg = deg_ref[0] + deg_ref[1] + 1.0
        dinv = lax.rsqrt(jnp.maximum(deg, 1.0))
        p = jnp.dot(x_ref[...], w1_ref[...], preferred_element_type=jnp.float32)
        g1_ref[...] = dinv * p

    return pl.pallas_call(
        body,
        grid=(n_pad // BM,),
        in_specs=[
            pl.BlockSpec((2, BM, 16), lambda i: (0, i, 0)),
            pl.BlockSpec((BM, k_pad), lambda i: (i, 0)),
            pl.BlockSpec((k_pad, 16), lambda i: (0, 0)),
        ],
        out_specs=pl.BlockSpec((BM, 16), lambda i: (i, 0)),
        out_shape=jax.ShapeDtypeStruct((n_pad, 16), jnp.float32),
    )


def _tc_layer2(n_pad):
    def body(deg_ref, s1p_ref, g1_ref, b1_ref, w2_ref, g2_ref):
        deg = deg_ref[0] + deg_ref[1] + 1.0
        dinv = lax.rsqrt(jnp.maximum(deg, 1.0))
        s1 = s1p_ref[0] + s1p_ref[1] + g1_ref[...]
        h = jnp.maximum(dinv * s1 + b1_ref[...], 0.0)
        p2 = jnp.dot(h, w2_ref[...], preferred_element_type=jnp.float32)
        g2_ref[...] = dinv[:, :1] * p2

    return pl.pallas_call(
        body,
        grid=(n_pad // BM,),
        in_specs=[
            pl.BlockSpec((2, BM, 16), lambda i: (0, i, 0)),
            pl.BlockSpec((2, BM, 16), lambda i: (0, i, 0)),
            pl.BlockSpec((BM, 16), lambda i: (i, 0)),
            pl.BlockSpec((1, 16), lambda i: (0, 0)),
            pl.BlockSpec((16, 64), lambda i: (0, 0)),
        ],
        out_specs=pl.BlockSpec((BM, 64), lambda i: (i, 0)),
        out_shape=jax.ShapeDtypeStruct((n_pad, 64), jnp.float32),
    )


def _tc_final(n_pad):
    def body(deg_ref, s2p_ref, g2_ref, b2_ref, out_ref):
        deg = deg_ref[0] + deg_ref[1] + 1.0
        dinv = lax.rsqrt(jnp.maximum(deg, 1.0))
        s2 = s2p_ref[0] + s2p_ref[1] + g2_ref[...]
        out_ref[...] = dinv[:, :1] * s2 + b2_ref[...]

    return pl.pallas_call(
        body,
        grid=(n_pad // BM,),
        in_specs=[
            pl.BlockSpec((2, BM, 16), lambda i: (0, i, 0)),
            pl.BlockSpec((2, BM, 64), lambda i: (0, i, 0)),
            pl.BlockSpec((BM, 64), lambda i: (i, 0)),
            pl.BlockSpec((1, 64), lambda i: (0, 0)),
        ],
        out_specs=pl.BlockSpec((BM, 64), lambda i: (i, 0)),
        out_shape=jax.ShapeDtypeStruct((n_pad, 64), jnp.float32),
    )


def kernel(x, edge_index, W1, b1, W2, b2):
    n, k = x.shape
    e = edge_index.shape[1]
    nc, ns = 2, 16
    nw = nc * ns

    n_pad = ((n + 16 + BM - 1) // BM) * BM
    k_pad = ((k + 127) // 128) * 128
    epw = ((e + nw * CHUNK - 1) // (nw * CHUNK)) * CHUNK
    nchunk = epw // CHUNK
    e_pad = epw * nw
    pad_cnt = e_pad - e

    pad_ar = jnp.arange(pad_cnt, dtype=jnp.int32)
    src = jnp.concatenate([edge_index[0], (pad_ar * 97) % n])
    dst = jnp.concatenate([edge_index[1], n + (pad_ar % 16)])
    src3 = src.reshape(nw, nchunk, CHUNK)
    dst3 = dst.reshape(nw, nchunk, CHUNK)

    xp = jnp.zeros((n_pad, k_pad), jnp.float32).at[:n, :k].set(x)
    w1p = jnp.zeros((k_pad, 16), jnp.float32).at[:k].set(W1)

    degp = _make_sc_scatter(n_pad, nchunk, 16, False, nc, ns)(dst3)
    g1 = _tc_layer1(n_pad, k_pad)(degp, xp, w1p)
    s1p = _make_sc_scatter(n_pad, nchunk, 16, True, nc, ns)(g1, src3, dst3)
    g2 = _tc_layer2(n_pad)(degp, s1p, g1, b1.reshape(1, 16), W2)
    s2p = _make_sc_scatter(n_pad, nchunk, 64, True, nc, ns)(g2, src3, dst3)
    out = _tc_final(n_pad)(degp, s2p, g2, b2.reshape(1, 64))
    return out[:n]


# speedup vs baseline: 26.5982x; 1.5904x over previous
import jax
import jax.numpy as jnp
from jax import lax
from jax.experimental import pallas as pl
from jax.experimental.pallas import tpu as pltpu
from jax.experimental.pallas import tpu_sc as plsc

CHUNK = 128
BM = 1000


def _make_sc_scatter(n_tbl, n_pad, nchunk, feat, with_gather, nc, ns):
    rows16 = n_pad // ns
    n_init = rows16 // CHUNK
    mesh = plsc.VectorSubcoreMesh(core_axis_name="c", subcore_axis_name="s")
    fpl = feat // 16

    def body(*refs):
        if with_gather:
            (table_hbm, src_hbm, dst_hbm, out_hbm,
             src_v, dst_v, rows_v, bounce_v, acc_sh, sems) = refs
        else:
            (dst_hbm, out_hbm,
             dst_v, rows_v, bounce_v, acc_sh, sems) = refs
        cid = lax.axis_index("c")
        sid = lax.axis_index("s")
        wid = sid * nc + cid

        zero = jnp.zeros((16,), jnp.float32)

        def zfill(i, carry):
            for jj in range(fpl):
                rows_v[0, i, pl.ds(jj * 16, 16)] = zero
            return carry

        lax.fori_loop(0, CHUNK, zfill, 0)
        for c in range(n_init):
            pltpu.sync_copy(rows_v.at[0],
                            acc_sh.at[pl.ds(sid * rows16 + c * CHUNK, CHUNK)])
        plsc.subcore_barrier()

        pltpu.sync_copy(dst_hbm.at[wid], dst_v)

        if with_gather:
            pltpu.sync_copy(src_hbm.at[wid], src_v)

            def gather_start(j, b):
                pltpu.async_copy(table_hbm.at[src_v.at[j]], rows_v.at[b],
                                 sems.at[b])

            def gather_wait(j, b):
                pltpu.make_async_copy(table_hbm.at[src_v.at[j]], rows_v.at[b],
                                      sems.at[b]).wait()

            gather_start(0, 0)
            gather_start(1, 1)

            def outer(i, carry):
                for b in range(2):
                    j = i * 2 + b
                    gather_wait(j, b)
                    pltpu.sync_copy(rows_v.at[b], acc_sh.at[dst_v.at[j]],
                                    add=True)

                    @pl.when(j + 2 < nchunk)
                    def _():
                        gather_start(j + 2, b)

                return carry

            lax.fori_loop(0, nchunk // 2, outer, 0)
        else:
            one = jnp.ones((16,), jnp.float32)

            def ofill(i, carry):
                for jj in range(fpl):
                    rows_v[0, i, pl.ds(jj * 16, 16)] = one
                return carry

            lax.fori_loop(0, CHUNK, ofill, 0)

            def edge_chunk(j, carry):
                pltpu.sync_copy(rows_v.at[0], acc_sh.at[dst_v.at[j]], add=True)
                return carry

            lax.fori_loop(0, nchunk, edge_chunk, 0)

        plsc.subcore_barrier()

        base = sid * rows16
        pltpu.sync_copy(acc_sh.at[pl.ds(base, rows16)], bounce_v)
        pltpu.sync_copy(bounce_v, out_hbm.at[cid, pl.ds(base, rows16)])

    scratch = [
        pltpu.VMEM((nchunk, CHUNK), jnp.int32),
        pltpu.VMEM((2, CHUNK, feat), jnp.float32),
        pltpu.VMEM((rows16, feat), jnp.float32),
        pltpu.VMEM_SHARED((n_pad, feat), jnp.float32),
        pltpu.SemaphoreType.DMA((2,)),
    ]
    if with_gather:
        scratch.insert(0, pltpu.VMEM((nchunk, CHUNK), jnp.int32))

    return pl.kernel(
        body,
        mesh=mesh,
        out_type=jax.ShapeDtypeStruct((nc, n_pad, feat), jnp.float32),
        scratch_types=scratch,
        compiler_params=pltpu.CompilerParams(use_tc_tiling_on_sc=False),
    )


def _tc_layer1(n, n_pad, k):
    def body(deg_ref, xt_ref, w1_ref, g1_ref):
        deg = deg_ref[0] + deg_ref[1] + 1.0
        dinv = lax.rsqrt(jnp.maximum(deg, 1.0))
        p = lax.dot_general(xt_ref[...], w1_ref[...],
                            (((0,), (0,)), ((), ())),
                            preferred_element_type=jnp.float32)
        g1_ref[...] = dinv[:n] * p

    return pl.pallas_call(
        body,
        out_shape=jax.ShapeDtypeStruct((n, 16), jnp.float32),
    )


def _tc_layer2(n, nb):
    def body(deg_ref, s1p_ref, g1_ref, b1_ref, w2_ref, g2_ref):
        deg = deg_ref[0] + deg_ref[1] + 1.0
        dinv = lax.rsqrt(jnp.maximum(deg, 1.0))
        s1 = s1p_ref[0] + s1p_ref[1] + g1_ref[...]
        h = jnp.maximum(dinv * s1 + b1_ref[...], 0.0)
        p2 = jnp.dot(h, w2_ref[...], preferred_element_type=jnp.float32)
        g2_ref[...] = dinv[:, :1] * p2

    return pl.pallas_call(
        body,
        grid=(nb,),
        in_specs=[
            pl.BlockSpec((2, BM, 16), lambda i: (0, i, 0)),
            pl.BlockSpec((2, BM, 16), lambda i: (0, i, 0)),
            pl.BlockSpec((BM, 16), lambda i: (i, 0)),
            pl.BlockSpec((1, 16), lambda i: (0, 0)),
            pl.BlockSpec((16, 64), lambda i: (0, 0)),
        ],
        out_specs=pl.BlockSpec((BM, 64), lambda i: (i, 0)),
        out_shape=jax.ShapeDtypeStruct((n, 64), jnp.float32),
    )


def _tc_final(n, nb):
    def body(deg_ref, s2p_ref, g2_ref, b2_ref, out_ref):
        deg = deg_ref[0] + deg_ref[1] + 1.0
        dinv = lax.rsqrt(jnp.maximum(deg, 1.0))
        s2 = s2p_ref[0] + s2p_ref[1] + g2_ref[...]
        out_ref[...] = dinv[:, :1] * s2 + b2_ref[...]

    return pl.pallas_call(
        body,
        grid=(nb,),
        in_specs=[
            pl.BlockSpec((2, BM, 16), lambda i: (0, i, 0)),
            pl.BlockSpec((2, BM, 64), lambda i: (0, i, 0)),
            pl.BlockSpec((BM, 64), lambda i: (i, 0)),
            pl.BlockSpec((1, 64), lambda i: (0, 0)),
        ],
        out_specs=pl.BlockSpec((BM, 64), lambda i: (i, 0)),
        out_shape=jax.ShapeDtypeStruct((n, 64), jnp.float32),
    )


def kernel(x, edge_index, W1, b1, W2, b2):
    n, k = x.shape
    e = edge_index.shape[1]
    nc, ns = 2, 16
    nw = nc * ns
    nb = n // BM

    n_pad = ((n + 16 + 255) // 256) * 256
    epw = ((e + nw * CHUNK - 1) // (nw * CHUNK)) * CHUNK
    nchunk = epw // CHUNK
    pad_cnt = epw * nw - e

    pad_ar = jnp.arange(pad_cnt, dtype=jnp.int32)
    src = jnp.concatenate([edge_index[0], (pad_ar * 97) % n])
    dst = jnp.concatenate([edge_index[1], n + (pad_ar % 16)])
    src3 = src.reshape(nw, nchunk, CHUNK)
    dst3 = dst.reshape(nw, nchunk, CHUNK)

    xt = jnp.swapaxes(x, 0, 1)

    degp = _make_sc_scatter(n, n_pad, nchunk, 16, False, nc, ns)(dst3)
    g1 = _tc_layer1(n, n_pad, k)(degp, xt, W1)
    s1p = _make_sc_scatter(n, n_pad, nchunk, 16, True, nc, ns)(g1, src3, dst3)
    g2 = _tc_layer2(n, nb)(degp, s1p, g1, b1.reshape(1, 16), W2)
    s2p = _make_sc_scatter(n, n_pad, nchunk, 64, True, nc, ns)(g2, src3, dst3)
    return _tc_final(n, nb)(degp, s2p, g2, b2.reshape(1, 64))


# speedup vs baseline: 28.4115x; 1.0682x over previous
import jax
import jax.numpy as jnp
from jax import lax
from jax.experimental import pallas as pl
from jax.experimental.pallas import tpu as pltpu
from jax.experimental.pallas import tpu_sc as plsc

CHUNK = 128
BM = 1000


def _make_sc_scatter(n_tbl, n_pad, nchunk, feat, with_gather, nc, ns):
    rows16 = n_pad // ns
    n_init = rows16 // CHUNK
    mesh = plsc.VectorSubcoreMesh(core_axis_name="c", subcore_axis_name="s")
    fpl = feat // 16

    def body(*refs):
        if with_gather:
            (table_hbm, src_hbm, dst_hbm, out_hbm,
             src_v, dst_v, rows_v, bounce_v, acc_sh, sems) = refs
        else:
            (dst_hbm, out_hbm,
             dst_v, rows_v, bounce_v, acc_sh, sems) = refs
        cid = lax.axis_index("c")
        sid = lax.axis_index("s")
        wid = sid * nc + cid

        idx_copy = pltpu.make_async_copy(dst_hbm.at[wid], dst_v, sems.at[0])
        idx_copy.start()
        if with_gather:
            src_copy = pltpu.make_async_copy(src_hbm.at[wid], src_v,
                                             sems.at[1])
            src_copy.start()

        zero = jnp.zeros((16,), jnp.float32)

        def zfill(i, carry):
            for jj in range(fpl):
                rows_v[0, i, pl.ds(jj * 16, 16)] = zero
            return carry

        lax.fori_loop(0, CHUNK, zfill, 0)
        for c in range(n_init):
            pltpu.sync_copy(rows_v.at[0],
                            acc_sh.at[pl.ds(sid * rows16 + c * CHUNK, CHUNK)])
        idx_copy.wait()
        if with_gather:
            src_copy.wait()
        plsc.subcore_barrier()

        if with_gather:
            def gather_start(j, b):
                pltpu.async_copy(table_hbm.at[src_v.at[j]], rows_v.at[b],
                                 sems.at[b])

            def gather_wait(j, b):
                pltpu.make_async_copy(table_hbm.at[src_v.at[j]], rows_v.at[b],
                                      sems.at[b]).wait()

            gather_start(0, 0)
            gather_start(1, 1)

            def outer(i, carry):
                for b in range(2):
                    j = i * 2 + b
                    gather_wait(j, b)
                    pltpu.sync_copy(rows_v.at[b], acc_sh.at[dst_v.at[j]],
                                    add=True)

                    @pl.when(j + 2 < nchunk)
                    def _():
                        gather_start(j + 2, b)

                return carry

            lax.fori_loop(0, nchunk // 2, outer, 0)
        else:
            one = jnp.ones((16,), jnp.float32)

            def ofill(i, carry):
                for jj in range(fpl):
                    rows_v[0, i, pl.ds(jj * 16, 16)] = one
                return carry

            lax.fori_loop(0, CHUNK, ofill, 0)

            def edge_chunk(j, carry):
                pltpu.sync_copy(rows_v.at[0], acc_sh.at[dst_v.at[j]], add=True)
                return carry

            lax.fori_loop(0, nchunk, edge_chunk, 0)

        plsc.subcore_barrier()

        base = sid * rows16
        pltpu.sync_copy(acc_sh.at[pl.ds(base, rows16)], bounce_v)
        pltpu.sync_copy(bounce_v, out_hbm.at[cid, pl.ds(base, rows16)])

    scratch = [
        pltpu.VMEM((nchunk, CHUNK), jnp.int32),
        pltpu.VMEM((2, CHUNK, feat), jnp.float32),
        pltpu.VMEM((rows16, feat), jnp.float32),
        pltpu.VMEM_SHARED((n_pad, feat), jnp.float32),
        pltpu.SemaphoreType.DMA((2,)),
    ]
    if with_gather:
        scratch.insert(0, pltpu.VMEM((nchunk, CHUNK), jnp.int32))

    return pl.kernel(
        body,
        mesh=mesh,
        out_type=jax.ShapeDtypeStruct((nc, n_pad, feat), jnp.float32),
        scratch_types=scratch,
        compiler_params=pltpu.CompilerParams(use_tc_tiling_on_sc=False),
    )


def _tc_layer1(n, n_pad, k):
    def body(deg_ref, xt_ref, w1t_ref, g1_ref, dinv_ref):
        deg = deg_ref[0] + deg_ref[1] + 1.0
        dinv = lax.rsqrt(jnp.maximum(deg, 1.0))[:n]
        p = lax.dot_general(xt_ref[...], w1t_ref[...],
                            (((0,), (1,)), ((), ())),
                            preferred_element_type=jnp.float32)
        g1_ref[...] = dinv * p
        dinv_ref[...] = dinv

    return pl.pallas_call(
        body,
        out_shape=(jax.ShapeDtypeStruct((n, 16), jnp.float32),
                   jax.ShapeDtypeStruct((n, 16), jnp.float32)),
    )


def _tc_layer2(n, nb):
    def body(dinv_ref, s1p_ref, g1_ref, b1_ref, w2t_ref, g2_ref):
        dinv = dinv_ref[...]
        s1 = s1p_ref[0] + s1p_ref[1] + g1_ref[...]
        h = jnp.maximum(dinv * s1 + b1_ref[...], 0.0)
        p2 = jnp.dot(h, w2t_ref[...], preferred_element_type=jnp.float32)
        g2_ref[...] = dinv[:, :1] * p2

    return pl.pallas_call(
        body,
        grid=(nb,),
        in_specs=[
            pl.BlockSpec((BM, 16), lambda i: (i, 0)),
            pl.BlockSpec((2, BM, 16), lambda i: (0, i, 0)),
            pl.BlockSpec((BM, 16), lambda i: (i, 0)),
            pl.BlockSpec((1, 16), lambda i: (0, 0)),
            pl.BlockSpec((16, 64), lambda i: (0, 0)),
        ],
        out_specs=pl.BlockSpec((BM, 64), lambda i: (i, 0)),
        out_shape=jax.ShapeDtypeStruct((n, 64), jnp.float32),
    )


def _tc_final(n):
    def body(dinv_ref, s2p_ref, g2_ref, b2_ref, out_ref):
        dinv = dinv_ref[...]
        s2 = s2p_ref[0][:n] + s2p_ref[1][:n] + g2_ref[...]
        out_ref[...] = (dinv[:, :1] * s2 + b2_ref[...]).T

    return pl.pallas_call(
        body,
        out_shape=jax.ShapeDtypeStruct((64, n), jnp.float32),
    )


def kernel(x, edge_index, W1, b1, W2, b2):
    n, k = x.shape
    e = edge_index.shape[1]
    nc, ns = 2, 16
    nw = nc * ns
    nb = n // BM

    n_pad = ((n + 16 + 255) // 256) * 256
    epw = ((e + nw * CHUNK - 1) // (nw * CHUNK)) * CHUNK
    nchunk = epw // CHUNK
    pad_cnt = epw * nw - e

    pad_ar = jnp.arange(pad_cnt, dtype=jnp.int32)
    src = jnp.concatenate([edge_index[0], pad_ar & 4095])
    dst = jnp.concatenate([edge_index[1], n + (pad_ar & 15)])
    src3 = src.reshape(nw, nchunk, CHUNK)
    dst3 = dst.reshape(nw, nchunk, CHUNK)

    xt = jnp.swapaxes(x, 0, 1)
    w1t = jnp.swapaxes(W1, 0, 1)

    degp = _make_sc_scatter(n, n_pad, nchunk, 16, False, nc, ns)(dst3)
    g1, dinv = _tc_layer1(n, n_pad, k)(degp, xt, w1t)
    s1p = _make_sc_scatter(n, n_pad, nchunk, 16, True, nc, ns)(g1, src3, dst3)
    g2 = _tc_layer2(n, nb)(dinv, s1p, g1, b1.reshape(1, 16), W2)
    s2p = _make_sc_scatter(n, n_pad, nchunk, 64, True, nc, ns)(g2, src3, dst3)
    out_t = _tc_final(n)(dinv, s2p, g2, b2.reshape(1, 64))
    return jnp.swapaxes(out_t, 0, 1)


# speedup vs baseline: 31.1048x; 1.0948x over previous
import jax
import jax.numpy as jnp
from jax import lax
from jax.experimental import pallas as pl
from jax.experimental.pallas import tpu as pltpu
from jax.experimental.pallas import tpu_sc as plsc

CHUNK = 128
BM = 1000
NBUF = 4
LOOK = 2


def _make_sc_scatter(n_tbl, n_pad, nchunk, feat, with_gather, nc, ns):
    rows16 = n_pad // ns
    n_init = rows16 // CHUNK
    mesh = plsc.VectorSubcoreMesh(core_axis_name="c", subcore_axis_name="s")
    fpl = feat // 16

    def body(*refs):
        if with_gather:
            (table_hbm, src_hbm, dst_hbm, out_hbm,
             src_v, dst_v, rows_v, bounce_v, acc_sh, gsems, ssems) = refs
        else:
            (dst_hbm, out_hbm,
             dst_v, rows_v, bounce_v, acc_sh, gsems, ssems) = refs
        cid = lax.axis_index("c")
        sid = lax.axis_index("s")
        wid = sid * nc + cid
        base = sid * rows16

        idx_copy = pltpu.make_async_copy(dst_hbm.at[wid], dst_v, gsems.at[0])
        idx_copy.start()
        if with_gather:
            src_copy = pltpu.make_async_copy(src_hbm.at[wid], src_v,
                                             gsems.at[1])
            src_copy.start()

        zero = jnp.zeros((16,), jnp.float32)

        def zfill(i, carry):
            for jj in range(fpl):
                rows_v[0, i, pl.ds(jj * 16, 16)] = zero
            return carry

        lax.fori_loop(0, CHUNK, zfill, 0)

        if with_gather:
            btile = n_tbl // rows16
            bfull = (n_tbl - btile * rows16) // CHUNK
            brem = n_tbl - btile * rows16 - bfull * CHUNK

            @pl.when(jnp.logical_and(cid == 0, sid < btile))
            def _():
                for c in range(n_init):
                    pltpu.sync_copy(
                        table_hbm.at[pl.ds(base + c * CHUNK, CHUNK)],
                        acc_sh.at[pl.ds(base + c * CHUNK, CHUNK)])

            @pl.when(jnp.logical_and(cid == 0, sid == btile))
            def _():
                b0 = btile * rows16
                for c in range(bfull):
                    pltpu.sync_copy(
                        table_hbm.at[pl.ds(b0 + c * CHUNK, CHUNK)],
                        acc_sh.at[pl.ds(b0 + c * CHUNK, CHUNK)])
                lo = b0 + bfull * CHUNK
                if brem:
                    pltpu.sync_copy(table_hbm.at[pl.ds(lo, brem)],
                                    acc_sh.at[pl.ds(lo, brem)])
                    pltpu.sync_copy(rows_v.at[0].at[pl.ds(0, CHUNK - brem)],
                                    acc_sh.at[pl.ds(lo + brem, CHUNK - brem)])
                for c in range(bfull + 1, n_init):
                    pltpu.sync_copy(
                        rows_v.at[0],
                        acc_sh.at[pl.ds(b0 + c * CHUNK, CHUNK)])

            @pl.when(jnp.logical_or(cid != 0, sid > btile))
            def _():
                for c in range(n_init):
                    pltpu.sync_copy(
                        rows_v.at[0],
                        acc_sh.at[pl.ds(base + c * CHUNK, CHUNK)])
        else:
            for c in range(n_init):
                pltpu.sync_copy(rows_v.at[0],
                                acc_sh.at[pl.ds(base + c * CHUNK, CHUNK)])

        idx_copy.wait()
        if with_gather:
            src_copy.wait()
        plsc.subcore_barrier()

        if with_gather:
            def gather_start(j, b):
                pltpu.async_copy(table_hbm.at[src_v.at[j]], rows_v.at[b],
                                 gsems.at[b])

            def gather_wait(j, b):
                pltpu.make_async_copy(table_hbm.at[src_v.at[j]], rows_v.at[b],
                                      gsems.at[b]).wait()

            def scatter_start(j, b):
                pltpu.make_async_copy(rows_v.at[b], acc_sh.at[dst_v.at[j]],
                                      ssems.at[b]).start(add=True)

            def scatter_wait(j, b):
                pltpu.make_async_copy(rows_v.at[b], acc_sh.at[dst_v.at[j]],
                                      ssems.at[b]).wait()

            for j0 in range(LOOK):
                gather_start(j0, j0 % NBUF)

            def outer(i, carry):
                for u in range(2):
                    j = i * 2 + u
                    b = j % NBUF
                    bg = (j + LOOK) % NBUF

                    @pl.when(j - (NBUF - LOOK) >= 0)
                    def _():
                        scatter_wait(j - (NBUF - LOOK), bg)

                    @pl.when(j + LOOK < nchunk)
                    def _():
                        gather_start(j + LOOK, bg)

                    gather_wait(j, b)
                    scatter_start(j, b)
                return carry

            lax.fori_loop(0, nchunk // 2, outer, 0)
            for t in range(LOOK, 0, -1):
                scatter_wait(nchunk - t, (nchunk - t) % NBUF)
        else:
            one = jnp.ones((16,), jnp.float32)

            def ofill(i, carry):
                for jj in range(fpl):
                    rows_v[0, i, pl.ds(jj * 16, 16)] = one
                return carry

            lax.fori_loop(0, CHUNK, ofill, 0)

            def s_start(j, carry):
                pltpu.make_async_copy(rows_v.at[0], acc_sh.at[dst_v.at[j]],
                                      ssems.at[0]).start(add=True)
                return carry

            def s_wait(j, carry):
                pltpu.make_async_copy(rows_v.at[0], acc_sh.at[dst_v.at[j]],
                                      ssems.at[0]).wait()
                return carry

            lax.fori_loop(0, nchunk, s_start, 0)
            lax.fori_loop(0, nchunk, s_wait, 0)

        plsc.subcore_barrier()

        pltpu.sync_copy(acc_sh.at[pl.ds(base, rows16)], bounce_v)
        pltpu.sync_copy(bounce_v, out_hbm.at[cid, pl.ds(base, rows16)])

    scratch = [
        pltpu.VMEM((nchunk, CHUNK), jnp.int32),
        pltpu.VMEM((NBUF, CHUNK, feat), jnp.float32),
        pltpu.VMEM((rows16, feat), jnp.float32),
        pltpu.VMEM_SHARED((n_pad, feat), jnp.float32),
        pltpu.SemaphoreType.DMA((NBUF,)),
        pltpu.SemaphoreType.DMA((NBUF,)),
    ]
    if with_gather:
        scratch.insert(0, pltpu.VMEM((nchunk, CHUNK), jnp.int32))

    return pl.kernel(
        body,
        mesh=mesh,
        out_type=jax.ShapeDtypeStruct((nc, n_pad, feat), jnp.float32),
        scratch_types=scratch,
        compiler_params=pltpu.CompilerParams(use_tc_tiling_on_sc=False),
    )


def _tc_layer1(n, n_pad, k):
    def body(deg_ref, xt_ref, w1t_ref, g1_ref, dinv_ref):
        deg = deg_ref[0] + deg_ref[1] + 1.0
        dinv = lax.rsqrt(jnp.maximum(deg, 1.0))[:n]
        p = lax.dot_general(xt_ref[...], w1t_ref[...],
                            (((0,), (1,)), ((), ())),
                            preferred_element_type=jnp.float32)
        g1_ref[...] = dinv * p
        dinv_ref[...] = dinv

    return pl.pallas_call(
        body,
        out_shape=(jax.ShapeDtypeStruct((n, 16), jnp.float32),
                   jax.ShapeDtypeStruct((n, 16), jnp.float32)),
    )


def _tc_layer2(n, nb):
    def body(dinv_ref, s1p_ref, b1_ref, w2_ref, g2_ref):
        dinv = dinv_ref[...]
        s1 = s1p_ref[0] + s1p_ref[1]
        h = jnp.maximum(dinv * s1 + b1_ref[...], 0.0)
        p2 = jnp.dot(h, w2_ref[...], preferred_element_type=jnp.float32)
        g2_ref[...] = dinv[:, :1] * p2

    return pl.pallas_call(
        body,
        grid=(nb,),
        in_specs=[
            pl.BlockSpec((BM, 16), lambda i: (i, 0)),
            pl.BlockSpec((2, BM, 16), lambda i: (0, i, 0)),
            pl.BlockSpec((1, 16), lambda i: (0, 0)),
            pl.BlockSpec((16, 64), lambda i: (0, 0)),
        ],
        out_specs=pl.BlockSpec((BM, 64), lambda i: (i, 0)),
        out_shape=jax.ShapeDtypeStruct((n, 64), jnp.float32),
    )


def _tc_final(n):
    def body(dinv_ref, s2p_ref, b2_ref, out_ref):
        dinv = dinv_ref[...]
        s2 = s2p_ref[0][:n] + s2p_ref[1][:n]
        out_ref[...] = (dinv[:, :1] * s2 + b2_ref[...]).T

    return pl.pallas_call(
        body,
        out_shape=jax.ShapeDtypeStruct((64, n), jnp.float32),
    )


def kernel(x, edge_index, W1, b1, W2, b2):
    n, k = x.shape
    e = edge_index.shape[1]
    nc, ns = 2, 16
    nw = nc * ns
    nb = n // BM

    n_pad = ((n + 16 + 255) // 256) * 256
    epw = ((e + nw * CHUNK - 1) // (nw * CHUNK)) * CHUNK
    nchunk = epw // CHUNK
    pad_cnt = epw * nw - e

    ei = edge_index.reshape(-1)
    pad_ar = jnp.arange(pad_cnt, dtype=jnp.int32)
    src = jnp.concatenate([ei[:e], pad_ar & 4095])
    dst = jnp.concatenate([ei[e:], n + (pad_ar & 15)])
    src3 = src.reshape(nw, nchunk, CHUNK)
    dst3 = dst.reshape(nw, nchunk, CHUNK)

    xt = jnp.swapaxes(x, 0, 1)
    w1t = jnp.swapaxes(W1, 0, 1)

    degp = _make_sc_scatter(n, n_pad, nchunk, 16, False, nc, ns)(dst3)
    g1, dinv = _tc_layer1(n, n_pad, k)(degp, xt, w1t)
    s1p = _make_sc_scatter(n, n_pad, nchunk, 16, True, nc, ns)(g1, src3, dst3)
    g2 = _tc_layer2(n, nb)(dinv, s1p, b1.reshape(1, 16), W2)
    s2p = _make_sc_scatter(n, n_pad, nchunk, 64, True, nc, ns)(g2, src3, dst3)
    out_t = _tc_final(n)(dinv, s2p, b2.reshape(1, 64))
    return jnp.swapaxes(out_t, 0, 1)
